# baseline (speedup 1.0000x reference)
import functools

import jax
import jax.numpy as jnp
from jax import lax
from jax.experimental import pallas as pl
from jax.experimental.pallas import tpu as pltpu
from jax.experimental.pallas import tpu_sc as plsc

F = 26
VOCAB = 100000
D = 32
B = 16384

NC = 2
NS = 16
NW = NC * NS
PAIRS = F * D
PER_W = PAIRS // NW
NBLK = PER_W // 2

VTH = 33408
VSZ2 = VOCAB - 2 * VTH
VBODY = VSZ2 - 32

_mesh = plsc.VectorSubcoreMesh(core_axis_name="c", subcore_axis_name="s")


@functools.partial(
    pl.kernel,
    mesh=_mesh,
    out_type=jax.ShapeDtypeStruct((F, D, B), jnp.float32),
    scratch_types=[
        pltpu.VMEM((VTH,), jnp.float32),
        pltpu.VMEM((VTH,), jnp.float32),
        pltpu.VMEM((B,), jnp.int32),
        pltpu.VMEM((B,), jnp.float32),
        pltpu.VMEM((B,), jnp.float32),
        pltpu.SemaphoreType.DMA,
        pltpu.SemaphoreType.DMA,
        pltpu.SemaphoreType.DMA,
        pltpu.SemaphoreType.DMA,
    ],
    compiler_params=pltpu.CompilerParams(needs_layout_passes=False),
)
def _lookup_kernel(x_hbm, tables_hbm, tail_hbm, out_hbm, row0, row1, idx_v,
                   ob0, ob1, r0, r1, w0, w1):
    wid = lax.axis_index("s") * NC + lax.axis_index("c")
    rows = [row0, row1]
    rsems = [r0, r1]
    outs = [ob0, ob1]
    wsems = [w0, w1]

    def fd(j):
        p = wid * PER_W + j
        return lax.shift_right_logical(p, 5), lax.bitwise_and(p, D - 1), p

    def load_parts(j, v, par):
        f, d, p = fd(j)
        if v < 2:
            return [(tables_hbm.at[f, d, pl.ds(v * VTH, VTH)],
                     rows[par].at[pl.ds(0, VTH)])]
        return [(tables_hbm.at[f, d, pl.ds(2 * VTH, VBODY)],
                 rows[par].at[pl.ds(0, VBODY)]),
                (tail_hbm.at[p], rows[par].at[pl.ds(VBODY, 128)])]

    def start_load(j, v, par):
        for src, dst in load_parts(j, v, par):
            pltpu.async_copy(src, dst, rsems[par])

    def wait_load(j, v, par):
        for src, dst in load_parts(j, v, par):
            pltpu.make_async_copy(src, dst, rsems[par]).wait()

    start_load(0, 0, 0)
    iota = lax.iota(jnp.int32, 16)

    def block(i, carry):
        for k in range(6):
            half = k // 3
            v = k % 3
            par = k % 2
            jj = 2 * i + half
            f, d, _ = fd(jj)

            if v == 0:
                fprev, _, _ = fd(lax.max(jj - 1, 0))

                @pl.when((jj == 0) | (f != fprev))
                def _(f=f):
                    pltpu.sync_copy(x_hbm.at[f], idx_v)

                @pl.when(jj >= 2)
                def _(half=half):
                    pltpu.make_async_copy(
                        outs[half], out_hbm.at[0, 0], wsems[half]).wait()

            if k < 5:
                nk = k + 1
                start_load(2 * i + nk // 3, nk % 3, nk % 2)
            else:
                @pl.when(i < NBLK - 1)
                def _():
                    start_load(2 * i + 2, 0, 0)

            wait_load(jj, v, par)
            row_v = rows[par]
            ob = outs[half]
            vsz = VTH if v < 2 else VSZ2

            @plsc.parallel_loop(0, B // 128, 1, unroll=2)
            def _(g, v=v, vsz=vsz, row_v=row_v, ob=ob):
                for u in range(8):
                    base = g * 128 + u * 16
                    sl = pl.ds(base, 16)
                    iv = idx_v[sl]
                    if v == 0:
                        ob[sl] = plsc.load_gather(
                            row_v, [lax.min(iv, VTH - 1)])
                    else:
                        local = iv - v * VTH
                        mask = (plsc.bitcast(local, jnp.uint32)
                                < jnp.uint32(vsz))
                        gv = plsc.load_gather(row_v, [local], mask=mask)
                        plsc.store_scatter(ob, [base + iota], gv, mask=mask)

            if v == 2:
                pltpu.async_copy(ob, out_hbm.at[f, d], wsems[half])
        return carry

    lax.fori_loop(0, NBLK, block, 0)
    for half in range(2):
        pltpu.make_async_copy(
            outs[half], out_hbm.at[0, 0], wsems[half]).wait()


def kernel(x, tables):
    x_t = jnp.swapaxes(x, 0, 1)
    tables_t = jnp.transpose(tables, (0, 2, 1))
    tail = jnp.transpose(tables[:, 2 * VTH + VBODY:, :], (0, 2, 1))
    tail = jnp.pad(tail.reshape(PAIRS, 32), ((0, 0), (0, 96)))
    out = _lookup_kernel(x_t, tables_t, tail)
    return jnp.transpose(out, (2, 0, 1))


# speedup vs baseline: 1.0493x; 1.0493x over previous
import functools

import jax
import jax.numpy as jnp
from jax import lax
from jax.experimental import pallas as pl
from jax.experimental.pallas import tpu as pltpu
from jax.experimental.pallas import tpu_sc as plsc

F = 26
VOCAB = 100000
D = 32
B = 16384

NC = 2
NS = 16
NW = NC * NS
PAIRS = F * D
PER_W = PAIRS // NW
NBLK = PER_W // 2

VTH = 33408
VSZ2 = VOCAB - 2 * VTH
VBODY = VSZ2 - 32

_mesh = plsc.VectorSubcoreMesh(core_axis_name="c", subcore_axis_name="s")


@functools.partial(
    pl.kernel,
    mesh=_mesh,
    out_type=jax.ShapeDtypeStruct((F, D, B), jnp.float32),
    scratch_types=[
        pltpu.VMEM((VTH,), jnp.float32),
        pltpu.VMEM((VTH,), jnp.float32),
        pltpu.VMEM((B,), jnp.int32),
        pltpu.VMEM((B,), jnp.float32),
        pltpu.VMEM((B,), jnp.float32),
        pltpu.SemaphoreType.DMA,
        pltpu.SemaphoreType.DMA,
        pltpu.SemaphoreType.DMA,
        pltpu.SemaphoreType.DMA,
    ],
    compiler_params=pltpu.CompilerParams(needs_layout_passes=False),
)
def _lookup_kernel(x_hbm, tables_hbm, tail_hbm, out_hbm, row0, row1, idx_v,
                   ob0, ob1, r0, r1, w0, w1):
    wid = lax.axis_index("s") * NC + lax.axis_index("c")
    rows = [row0, row1]
    rsems = [r0, r1]
    outs = [ob0, ob1]
    wsems = [w0, w1]

    def fd(j):
        p = wid * PER_W + j
        return lax.shift_right_logical(p, 5), lax.bitwise_and(p, D - 1), p

    def load_parts(j, v, par):
        f, d, p = fd(j)
        if v < 2:
            return [(tables_hbm.at[f, d, pl.ds(v * VTH, VTH)],
                     rows[par].at[pl.ds(0, VTH)])]
        return [(tables_hbm.at[f, d, pl.ds(2 * VTH, VBODY)],
                 rows[par].at[pl.ds(0, VBODY)]),
                (tail_hbm.at[p], rows[par].at[pl.ds(VBODY, 128)])]

    def start_load(j, v, par):
        for src, dst in load_parts(j, v, par):
            pltpu.async_copy(src, dst, rsems[par])

    def wait_load(j, v, par):
        for src, dst in load_parts(j, v, par):
            pltpu.make_async_copy(src, dst, rsems[par]).wait()

    start_load(0, 0, 0)
    iota = lax.iota(jnp.int32, 16)

    def block(i, carry):
        for k in range(6):
            half = k // 3
            v = k % 3
            par = k % 2
            jj = 2 * i + half
            f, d, _ = fd(jj)

            if v == 0:
                fprev, _, _ = fd(lax.max(jj - 1, 0))

                @pl.when((jj == 0) | (f != fprev))
                def _(f=f):
                    pltpu.sync_copy(x_hbm.at[f], idx_v)

                @pl.when(jj >= 2)
                def _(half=half):
                    pltpu.make_async_copy(
                        outs[half], out_hbm.at[0, 0], wsems[half]).wait()

            if k < 5:
                nk = k + 1
                start_load(2 * i + nk // 3, nk % 3, nk % 2)
            else:
                @pl.when(i < NBLK - 1)
                def _():
                    start_load(2 * i + 2, 0, 0)

            wait_load(jj, v, par)
            row_v = rows[par]
            ob = outs[half]
            vsz = VTH if v < 2 else VSZ2

            @plsc.parallel_loop(0, B // 128, 1, unroll=4)
            def _(g, v=v, vsz=vsz, row_v=row_v, ob=ob):
                for u in range(8):
                    base = g * 128 + u * 16
                    sl = pl.ds(base, 16)
                    iv = idx_v[sl]
                    local = iv if v == 0 else iv - v * VTH
                    mask = plsc.bitcast(local, jnp.uint32) < jnp.uint32(vsz)
                    gv = plsc.load_gather(row_v, [local], mask=mask)
                    plsc.store_scatter(ob, [base + iota], gv, mask=mask)

            if v == 2:
                pltpu.async_copy(ob, out_hbm.at[f, d], wsems[half])
        return carry

    lax.fori_loop(0, NBLK, block, 0)
    for half in range(2):
        pltpu.make_async_copy(
            outs[half], out_hbm.at[0, 0], wsems[half]).wait()


def kernel(x, tables):
    x_t = jnp.swapaxes(x, 0, 1)
    tables_t = jnp.transpose(tables, (0, 2, 1))
    tail = jnp.transpose(tables[:, 2 * VTH + VBODY:, :], (0, 2, 1))
    tail = jnp.pad(tail.reshape(PAIRS, 32), ((0, 0), (0, 96)))
    out = _lookup_kernel(x_t, tables_t, tail)
    return jnp.transpose(out, (2, 0, 1))
